# Initial kernel scaffold; baseline (speedup 1.0000x reference)
import functools
import math

import jax
import jax.numpy as jnp
from jax import lax
from jax.experimental import pallas as pl
from jax.experimental.pallas import tpu as pltpu
from jax.experimental.pallas import tpu_sc as plsc

INPUT_DIM = 64
NUM_PROJS = 8
BLOCK_SIZE = 256
SAMPLE_SIZE = 256

NC = 2
NS = 16
LANES = 16
CHUNK = 128
NBUF = 4


def _hash_body(q_ref, k_ref, proj_ref, cq_ref, ck_ref):
    proj = proj_ref[...]
    powers = (2 ** lax.iota(jnp.int32, NUM_PROJS))[None, :]

    def codes(x):
        s = lax.dot_general(x, proj, (((1,), (0,)), ((), ())),
                            preferred_element_type=jnp.float32)
        bits = (s > 0).astype(jnp.int32)
        return jnp.sum(bits * powers, axis=-1)

    cq_ref[0, 0] = codes(q_ref[0])
    ck_ref[0, 0] = codes(k_ref[0])


def _hash_codes(qf, kf, proj):
    BH, S, D = qf.shape
    return pl.pallas_call(
        _hash_body,
        grid=(BH,),
        in_specs=[
            pl.BlockSpec((1, S, D), lambda c: (c, 0, 0)),
            pl.BlockSpec((1, S, D), lambda c: (c, 0, 0)),
            pl.BlockSpec((D, NUM_PROJS), lambda c: (0, 0)),
        ],
        out_specs=[
            pl.BlockSpec((1, 1, S), lambda c: (c, 0, 0)),
            pl.BlockSpec((1, 1, S), lambda c: (c, 0, 0)),
        ],
        out_shape=[
            jax.ShapeDtypeStruct((BH, 1, S), jnp.int32),
            jax.ShapeDtypeStruct((BH, 1, S), jnp.int32),
        ],
    )(qf, kf, proj)


def _count_sort(cvm, hist, cur, rvm, svm, n):
    per_lane = n // LANES
    lane = lax.iota(jnp.int32, LANES)
    ones = jnp.ones((LANES,), jnp.int32)
    zeros = jnp.zeros((LANES,), jnp.int32)

    def zero_body(t, _):
        plsc.store_scatter(hist, [t * LANES + lane], zeros)
        return 0

    lax.fori_loop(0, 256, zero_body, 0)

    def p1_body(t, _):
        idx = lane * per_lane + t
        c16 = plsc.load_gather(cvm, [idx])
        plsc.addupdate_scatter(hist, [c16 * LANES + lane], ones)
        return 0

    lax.fori_loop(0, per_lane, p1_body, 0)

    def p2_body(c, carry):
        v = plsc.load_gather(hist, [c * LANES + lane])
        cs = plsc.cumsum(v)
        plsc.store_scatter(cur, [c * LANES + lane], carry + cs - v)
        return carry + jnp.sum(v)

    lax.fori_loop(0, 256, p2_body, jnp.int32(0))

    def p3_body(t, _):
        idx = lane * per_lane + t
        c16 = plsc.load_gather(cvm, [idx])
        slot = c16 * LANES + lane
        r16 = plsc.load_gather(cur, [slot])
        if rvm is not None:
            plsc.store_scatter(rvm, [idx], r16)
        plsc.store_scatter(svm, [r16], idx)
        plsc.addupdate_scatter(cur, [slot], ones)
        return 0

    lax.fori_loop(0, per_lane, p3_body, 0)


def _gather_rows(table, idxvm, outdst, chunks, bufs, sems_r, sems_w):
    nb = min(NBUF, chunks)
    steps = chunks // nb

    def step(s, _):
        handles = []
        for b in range(nb):
            c = s * nb + b
            idxr = idxvm.at[pl.ds(c * CHUNK, CHUNK)]
            handles.append(pltpu.async_copy(table.at[idxr], bufs.at[b],
                                            sems_r[b]))
        writes = []
        for b in range(nb):
            handles[b].wait()
            c = s * nb + b
            writes.append(pltpu.async_copy(
                bufs.at[b], outdst.at[pl.ds(c * CHUNK, CHUNK)], sems_w[b]))
        for wh in writes:
            wh.wait()
        return 0

    lax.fori_loop(0, steps, step, 0)


def _sort_gather(codes_q, codes_k, qf, kf, vf, sampled):
    BH, S, D = qf.shape
    mesh = plsc.VectorSubcoreMesh(core_axis_name="c", subcore_axis_name="s",
                                  num_cores=NC, num_subcores=NS)

    @functools.partial(
        pl.kernel,
        out_type=[
            jax.ShapeDtypeStruct((BH, S, D), jnp.float32),
            jax.ShapeDtypeStruct((BH, S, D), jnp.float32),
            jax.ShapeDtypeStruct((BH, S, D), jnp.float32),
            jax.ShapeDtypeStruct((BH, SAMPLE_SIZE, D), jnp.float32),
            jax.ShapeDtypeStruct((BH, SAMPLE_SIZE, D), jnp.float32),
            jax.ShapeDtypeStruct((BH, S), jnp.int32),
            jax.ShapeDtypeStruct((BH, S), jnp.int32),
        ],
        mesh=mesh,
        scratch_types=[
            pltpu.VMEM((S,), jnp.int32),
            pltpu.VMEM((256 * LANES,), jnp.int32),
            pltpu.VMEM((256 * LANES,), jnp.int32),
            pltpu.VMEM((S,), jnp.int32),
            pltpu.VMEM((S,), jnp.int32),
            pltpu.VMEM((S,), jnp.int32),
            pltpu.VMEM((SAMPLE_SIZE,), jnp.int32),
            pltpu.VMEM((NBUF, CHUNK, D), jnp.float32),
            pltpu.SemaphoreType.DMA,
            pltpu.SemaphoreType.DMA,
            pltpu.SemaphoreType.DMA,
            pltpu.SemaphoreType.DMA,
            pltpu.SemaphoreType.DMA,
            pltpu.SemaphoreType.DMA,
            pltpu.SemaphoreType.DMA,
            pltpu.SemaphoreType.DMA,
        ],
    )
    def run(cq_hbm, ck_hbm, q_hbm, k_hbm, v_hbm, smp_hbm,
            qs_o, ks_o, vs_o, ksub_o, vsub_o, sidx_o, rank_o,
            cvm, hist, cur, rvm, svq, svk, smp, bufs,
            sr0, sr1, sr2, sr3, sw0, sw1, sw2, sw3):
        sems_r = [sr0, sr1, sr2, sr3]
        sems_w = [sw0, sw1, sw2, sw3]
        w = lax.axis_index("s") * NC + lax.axis_index("c")

        pltpu.sync_copy(cq_hbm.at[w], cvm)
        _count_sort(cvm, hist, cur, rvm, svq, S)
        pltpu.sync_copy(rvm, rank_o.at[w])
        pltpu.sync_copy(svq, sidx_o.at[w])

        pltpu.sync_copy(ck_hbm.at[w], cvm)
        _count_sort(cvm, hist, cur, None, svk, S)

        pltpu.sync_copy(smp_hbm.at[w], smp)

        nchunks = S // CHUNK
        _gather_rows(q_hbm.at[w], svq, qs_o.at[w], nchunks, bufs, sems_r,
                     sems_w)
        _gather_rows(k_hbm.at[w], svk, ks_o.at[w], nchunks, bufs, sems_r,
                     sems_w)
        _gather_rows(v_hbm.at[w], svk, vs_o.at[w], nchunks, bufs, sems_r,
                     sems_w)
        _gather_rows(k_hbm.at[w], smp, ksub_o.at[w], SAMPLE_SIZE // CHUNK,
                     bufs, sems_r, sems_w)
        _gather_rows(v_hbm.at[w], smp, vsub_o.at[w], SAMPLE_SIZE // CHUNK,
                     bufs, sems_r, sems_w)

    return run(codes_q, codes_k, qf, kf, vf, sampled)


def _unsort(attn_sorted, rank):
    BH, S, D = attn_sorted.shape
    mesh = plsc.VectorSubcoreMesh(core_axis_name="c", subcore_axis_name="s",
                                  num_cores=NC, num_subcores=NS)

    @functools.partial(
        pl.kernel,
        out_type=jax.ShapeDtypeStruct((BH, S, D), jnp.float32),
        mesh=mesh,
        scratch_types=[
            pltpu.VMEM((S,), jnp.int32),
            pltpu.VMEM((NBUF, CHUNK, D), jnp.float32),
            pltpu.SemaphoreType.DMA,
            pltpu.SemaphoreType.DMA,
            pltpu.SemaphoreType.DMA,
            pltpu.SemaphoreType.DMA,
            pltpu.SemaphoreType.DMA,
            pltpu.SemaphoreType.DMA,
            pltpu.SemaphoreType.DMA,
            pltpu.SemaphoreType.DMA,
        ],
    )
    def run(a_hbm, r_hbm, out_hbm, rvm, bufs,
            sr0, sr1, sr2, sr3, sw0, sw1, sw2, sw3):
        w = lax.axis_index("s") * NC + lax.axis_index("c")
        pltpu.sync_copy(r_hbm.at[w], rvm)
        _gather_rows(a_hbm.at[w], rvm, out_hbm.at[w], S // CHUNK, bufs,
                     [sr0, sr1, sr2, sr3], [sw0, sw1, sw2, sw3])

    return run(attn_sorted, rank)


def _attn_body(scale, q_ref, k_ref, v_ref, ks_ref, vs_ref, sset_ref,
               sidx_ref, out_ref):
    q = q_ref[0]
    k = k_ref[0]
    v = v_ref[0]

    s1 = lax.dot_general(q, k, (((1,), (1,)), ((), ())),
                         preferred_element_type=jnp.float32) * scale
    m1 = jnp.max(s1, axis=-1, keepdims=True)
    e1 = jnp.exp(s1 - m1)
    l1 = jnp.sum(e1, axis=-1, keepdims=True)
    o1 = lax.dot_general(e1, v, (((1,), (0,)), ((), ())),
                         preferred_element_type=jnp.float32) / l1
    lse1 = m1 + jnp.log(l1)

    ks = ks_ref[0]
    vs = vs_ref[0]
    qpos = sidx_ref[0]
    spos = sset_ref[0]
    mask = (qpos // BLOCK_SIZE) == (spos // BLOCK_SIZE)
    bias = jnp.where(mask, jnp.finfo(jnp.float32).min, jnp.float32(0.0))
    s2 = lax.dot_general(q, ks, (((1,), (1,)), ((), ())),
                         preferred_element_type=jnp.float32) * scale + bias
    m2 = jnp.max(s2, axis=-1, keepdims=True)
    e2 = jnp.exp(s2 - m2)
    l2 = jnp.sum(e2, axis=-1, keepdims=True)
    o2 = lax.dot_general(e2, vs, (((1,), (0,)), ((), ())),
                         preferred_element_type=jnp.float32) / l2
    lse2 = m2 + jnp.log(l2) + math.log(16.0)

    c = 1.0 / (1.0 + jnp.exp(lse2 - lse1))
    out_ref[0] = c * o1 + (1.0 - c) * o2


def _block_attention(q_s, k_s, v_s, ksub, vsub, sset3, sidx3, scale):
    BH, S, D = q_s.shape
    nb = S // BLOCK_SIZE
    return pl.pallas_call(
        functools.partial(_attn_body, scale),
        grid=(BH * nb,),
        in_specs=[
            pl.BlockSpec((1, BLOCK_SIZE, D), lambda c: (c // nb, c % nb, 0)),
            pl.BlockSpec((1, BLOCK_SIZE, D), lambda c: (c // nb, c % nb, 0)),
            pl.BlockSpec((1, BLOCK_SIZE, D), lambda c: (c // nb, c % nb, 0)),
            pl.BlockSpec((1, SAMPLE_SIZE, D), lambda c: (c // nb, 0, 0)),
            pl.BlockSpec((1, SAMPLE_SIZE, D), lambda c: (c // nb, 0, 0)),
            pl.BlockSpec((1, 1, SAMPLE_SIZE), lambda c: (c // nb, 0, 0)),
            pl.BlockSpec((1, BLOCK_SIZE, 1), lambda c: (c, 0, 0)),
        ],
        out_specs=pl.BlockSpec((1, BLOCK_SIZE, D),
                               lambda c: (c // nb, c % nb, 0)),
        out_shape=jax.ShapeDtypeStruct((BH, S, D), jnp.float32),
    )(q_s, k_s, v_s, ksub, vsub, sset3, sidx3)


def kernel(query, key, value, proj_dir):
    B, H, S, D = query.shape
    BH = B * H
    scale = D ** (-0.5)

    qf = query.reshape(BH, S, D)
    kf = key.reshape(BH, S, D)
    vf = value.reshape(BH, S, D)
    proj = proj_dir[0, 0]

    codes_q, codes_k = _hash_codes(qf, kf, proj)
    codes_q = codes_q.reshape(BH, S)
    codes_k = codes_k.reshape(BH, S)

    skey = jax.random.key(42)
    sampled = jax.random.randint(skey, (B, H, SAMPLE_SIZE), 0, S)
    sampled = sampled.reshape(BH, SAMPLE_SIZE).astype(jnp.int32)

    q_s, k_s, v_s, ksub, vsub, sidx, rank = _sort_gather(
        codes_q, codes_k, qf, kf, vf, sampled)

    attn_sorted = _block_attention(
        q_s, k_s, v_s, ksub, vsub,
        sampled.reshape(BH, 1, SAMPLE_SIZE),
        sidx.reshape(BH * (S // BLOCK_SIZE), BLOCK_SIZE, 1),
        scale)

    out = _unsort(attn_sorted, rank)
    return out.reshape(B, H, S, D)


# speedup vs baseline: 6.6505x; 6.6505x over previous
import functools
import math

import jax
import jax.numpy as jnp
from jax import lax
from jax.experimental import pallas as pl
from jax.experimental.pallas import tpu as pltpu
from jax.experimental.pallas import tpu_sc as plsc

INPUT_DIM = 64
NUM_PROJS = 8
BLOCK_SIZE = 256
SAMPLE_SIZE = 256

NC = 2
NS = 16
LANES = 16
CHUNK = 128
NBUF = 4


def _hash_body(q_ref, k_ref, proj_ref, cq_ref, ck_ref):
    proj = proj_ref[...]
    powers = (2 ** lax.iota(jnp.int32, NUM_PROJS))[None, :]

    def codes(x):
        s = lax.dot_general(x, proj, (((1,), (0,)), ((), ())),
                            preferred_element_type=jnp.float32)
        bits = (s > 0).astype(jnp.int32)
        return jnp.sum(bits * powers, axis=-1)

    cq_ref[0, 0] = codes(q_ref[0])
    ck_ref[0, 0] = codes(k_ref[0])


def _hash_codes(qf, kf, proj):
    BH, S, D = qf.shape
    return pl.pallas_call(
        _hash_body,
        grid=(BH,),
        in_specs=[
            pl.BlockSpec((1, S, D), lambda c: (c, 0, 0)),
            pl.BlockSpec((1, S, D), lambda c: (c, 0, 0)),
            pl.BlockSpec((D, NUM_PROJS), lambda c: (0, 0)),
        ],
        out_specs=[
            pl.BlockSpec((1, 1, S), lambda c: (c, 0, 0)),
            pl.BlockSpec((1, 1, S), lambda c: (c, 0, 0)),
        ],
        out_shape=[
            jax.ShapeDtypeStruct((BH, 1, S), jnp.int32),
            jax.ShapeDtypeStruct((BH, 1, S), jnp.int32),
        ],
    )(qf, kf, proj)


def _count_sort(cvm, hist, cur, rvm, svm, n):
    per_lane = n // LANES
    lane = lax.iota(jnp.int32, LANES)
    ones = jnp.ones((LANES,), jnp.int32)
    zeros = jnp.zeros((LANES,), jnp.int32)

    def zero_body(t, _):
        plsc.store_scatter(hist, [t * LANES + lane], zeros)
        return 0

    lax.fori_loop(0, 256, zero_body, 0)

    def p1_body(t, _):
        idx = lane * per_lane + t
        c16 = plsc.load_gather(cvm, [idx])
        plsc.addupdate_scatter(hist, [c16 * LANES + lane], ones)
        return 0

    lax.fori_loop(0, per_lane, p1_body, 0)

    def p2_body(c, carry):
        v = plsc.load_gather(hist, [c * LANES + lane])
        cs = plsc.cumsum(v)
        plsc.store_scatter(cur, [c * LANES + lane], carry + cs - v)
        return carry + jnp.sum(v)

    lax.fori_loop(0, 256, p2_body, jnp.int32(0))

    def p3_body(t, _):
        idx = lane * per_lane + t
        c16 = plsc.load_gather(cvm, [idx])
        slot = c16 * LANES + lane
        r16 = plsc.load_gather(cur, [slot])
        if rvm is not None:
            plsc.store_scatter(rvm, [idx], r16)
        plsc.store_scatter(svm, [r16], idx)
        plsc.addupdate_scatter(cur, [slot], ones)
        return 0

    lax.fori_loop(0, per_lane, p3_body, 0)


def _gather_rows(table, idxvm, outdst, chunks, bufs, sems_r, sems_w):
    nb = min(NBUF, chunks)
    steps = chunks // nb

    def step(s, _):
        handles = []
        for b in range(nb):
            c = s * nb + b
            idxr = idxvm.at[pl.ds(c * CHUNK, CHUNK)]
            handles.append(pltpu.async_copy(table.at[idxr], bufs.at[b],
                                            sems_r[b]))
        writes = []
        for b in range(nb):
            handles[b].wait()
            c = s * nb + b
            writes.append(pltpu.async_copy(
                bufs.at[b], outdst.at[pl.ds(c * CHUNK, CHUNK)], sems_w[b]))
        for wh in writes:
            wh.wait()
        return 0

    lax.fori_loop(0, steps, step, 0)


def _sort_gather(codes_q, codes_k, qf, kf, vf, sampled):
    BH, S, D = qf.shape
    mesh = plsc.VectorSubcoreMesh(core_axis_name="c", subcore_axis_name="s",
                                  num_cores=NC, num_subcores=NS)

    @functools.partial(
        pl.kernel,
        out_type=[
            jax.ShapeDtypeStruct((BH, S, D), jnp.float32),
            jax.ShapeDtypeStruct((BH, S, D), jnp.float32),
            jax.ShapeDtypeStruct((BH, S, D), jnp.float32),
            jax.ShapeDtypeStruct((BH, SAMPLE_SIZE, D), jnp.float32),
            jax.ShapeDtypeStruct((BH, SAMPLE_SIZE, D), jnp.float32),
            jax.ShapeDtypeStruct((BH, S), jnp.int32),
            jax.ShapeDtypeStruct((BH, S), jnp.int32),
        ],
        mesh=mesh,
        scratch_types=[
            pltpu.VMEM((S,), jnp.int32),
            pltpu.VMEM((256 * LANES,), jnp.int32),
            pltpu.VMEM((256 * LANES,), jnp.int32),
            pltpu.VMEM((S,), jnp.int32),
            pltpu.VMEM((S,), jnp.int32),
            pltpu.VMEM((S,), jnp.int32),
            pltpu.VMEM((SAMPLE_SIZE,), jnp.int32),
            pltpu.VMEM((NBUF, CHUNK, D), jnp.float32),
            pltpu.SemaphoreType.DMA,
            pltpu.SemaphoreType.DMA,
            pltpu.SemaphoreType.DMA,
            pltpu.SemaphoreType.DMA,
            pltpu.SemaphoreType.DMA,
            pltpu.SemaphoreType.DMA,
            pltpu.SemaphoreType.DMA,
            pltpu.SemaphoreType.DMA,
        ],
        compiler_params=pltpu.CompilerParams(needs_layout_passes=False, use_tc_tiling_on_sc=False),
    )
    def run(cq_hbm, ck_hbm, q_hbm, k_hbm, v_hbm, smp_hbm,
            qs_o, ks_o, vs_o, ksub_o, vsub_o, sidx_o, rank_o,
            cvm, hist, cur, rvm, svq, svk, smp, bufs,
            sr0, sr1, sr2, sr3, sw0, sw1, sw2, sw3):
        sems_r = [sr0, sr1, sr2, sr3]
        sems_w = [sw0, sw1, sw2, sw3]
        w = lax.axis_index("s") * NC + lax.axis_index("c")

        pltpu.sync_copy(cq_hbm.at[w], cvm)
        _count_sort(cvm, hist, cur, rvm, svq, S)
        pltpu.sync_copy(rvm, rank_o.at[w])
        pltpu.sync_copy(svq, sidx_o.at[w])

        pltpu.sync_copy(ck_hbm.at[w], cvm)
        _count_sort(cvm, hist, cur, None, svk, S)

        pltpu.sync_copy(smp_hbm.at[w], smp)

        nchunks = S // CHUNK
        _gather_rows(q_hbm.at[w], svq, qs_o.at[w], nchunks, bufs, sems_r,
                     sems_w)
        _gather_rows(k_hbm.at[w], svk, ks_o.at[w], nchunks, bufs, sems_r,
                     sems_w)
        _gather_rows(v_hbm.at[w], svk, vs_o.at[w], nchunks, bufs, sems_r,
                     sems_w)
        _gather_rows(k_hbm.at[w], smp, ksub_o.at[w], SAMPLE_SIZE // CHUNK,
                     bufs, sems_r, sems_w)
        _gather_rows(v_hbm.at[w], smp, vsub_o.at[w], SAMPLE_SIZE // CHUNK,
                     bufs, sems_r, sems_w)

    return run(codes_q, codes_k, qf, kf, vf, sampled)


def _unsort(attn_sorted, rank):
    BH, S, D = attn_sorted.shape
    mesh = plsc.VectorSubcoreMesh(core_axis_name="c", subcore_axis_name="s",
                                  num_cores=NC, num_subcores=NS)

    @functools.partial(
        pl.kernel,
        out_type=jax.ShapeDtypeStruct((BH, S, D), jnp.float32),
        mesh=mesh,
        scratch_types=[
            pltpu.VMEM((S,), jnp.int32),
            pltpu.VMEM((NBUF, CHUNK, D), jnp.float32),
            pltpu.SemaphoreType.DMA,
            pltpu.SemaphoreType.DMA,
            pltpu.SemaphoreType.DMA,
            pltpu.SemaphoreType.DMA,
            pltpu.SemaphoreType.DMA,
            pltpu.SemaphoreType.DMA,
            pltpu.SemaphoreType.DMA,
            pltpu.SemaphoreType.DMA,
        ],
        compiler_params=pltpu.CompilerParams(needs_layout_passes=False, use_tc_tiling_on_sc=False),
    )
    def run(a_hbm, r_hbm, out_hbm, rvm, bufs,
            sr0, sr1, sr2, sr3, sw0, sw1, sw2, sw3):
        w = lax.axis_index("s") * NC + lax.axis_index("c")
        pltpu.sync_copy(r_hbm.at[w], rvm)
        _gather_rows(a_hbm.at[w], rvm, out_hbm.at[w], S // CHUNK, bufs,
                     [sr0, sr1, sr2, sr3], [sw0, sw1, sw2, sw3])

    return run(attn_sorted, rank)


def _attn_body(scale, q_ref, k_ref, v_ref, ks_ref, vs_ref, sset_ref,
               sidx_ref, out_ref):
    q = q_ref[0]
    k = k_ref[0]
    v = v_ref[0]

    s1 = lax.dot_general(q, k, (((1,), (1,)), ((), ())),
                         preferred_element_type=jnp.float32) * scale
    m1 = jnp.max(s1, axis=-1, keepdims=True)
    e1 = jnp.exp(s1 - m1)
    l1 = jnp.sum(e1, axis=-1, keepdims=True)
    o1 = lax.dot_general(e1, v, (((1,), (0,)), ((), ())),
                         preferred_element_type=jnp.float32) / l1
    lse1 = m1 + jnp.log(l1)

    ks = ks_ref[0]
    vs = vs_ref[0]
    qpos = sidx_ref[0]
    spos = sset_ref[0]
    mask = (qpos // BLOCK_SIZE) == (spos // BLOCK_SIZE)
    bias = jnp.where(mask, jnp.finfo(jnp.float32).min, jnp.float32(0.0))
    s2 = lax.dot_general(q, ks, (((1,), (1,)), ((), ())),
                         preferred_element_type=jnp.float32) * scale + bias
    m2 = jnp.max(s2, axis=-1, keepdims=True)
    e2 = jnp.exp(s2 - m2)
    l2 = jnp.sum(e2, axis=-1, keepdims=True)
    o2 = lax.dot_general(e2, vs, (((1,), (0,)), ((), ())),
                         preferred_element_type=jnp.float32) / l2
    lse2 = m2 + jnp.log(l2) + math.log(16.0)

    c = 1.0 / (1.0 + jnp.exp(lse2 - lse1))
    out_ref[0] = c * o1 + (1.0 - c) * o2


def _block_attention(q_s, k_s, v_s, ksub, vsub, sset3, sidx3, scale):
    BH, S, D = q_s.shape
    nb = S // BLOCK_SIZE
    return pl.pallas_call(
        functools.partial(_attn_body, scale),
        grid=(BH * nb,),
        in_specs=[
            pl.BlockSpec((1, BLOCK_SIZE, D), lambda c: (c // nb, c % nb, 0)),
            pl.BlockSpec((1, BLOCK_SIZE, D), lambda c: (c // nb, c % nb, 0)),
            pl.BlockSpec((1, BLOCK_SIZE, D), lambda c: (c // nb, c % nb, 0)),
            pl.BlockSpec((1, SAMPLE_SIZE, D), lambda c: (c // nb, 0, 0)),
            pl.BlockSpec((1, SAMPLE_SIZE, D), lambda c: (c // nb, 0, 0)),
            pl.BlockSpec((1, 1, SAMPLE_SIZE), lambda c: (c // nb, 0, 0)),
            pl.BlockSpec((1, BLOCK_SIZE, 1), lambda c: (c, 0, 0)),
        ],
        out_specs=pl.BlockSpec((1, BLOCK_SIZE, D),
                               lambda c: (c // nb, c % nb, 0)),
        out_shape=jax.ShapeDtypeStruct((BH, S, D), jnp.float32),
    )(q_s, k_s, v_s, ksub, vsub, sset3, sidx3)


def kernel(query, key, value, proj_dir):
    B, H, S, D = query.shape
    BH = B * H
    scale = D ** (-0.5)

    qf = query.reshape(BH, S, D)
    kf = key.reshape(BH, S, D)
    vf = value.reshape(BH, S, D)
    proj = proj_dir[0, 0]

    codes_q, codes_k = _hash_codes(qf, kf, proj)
    codes_q = codes_q.reshape(BH, S)
    codes_k = codes_k.reshape(BH, S)

    skey = jax.random.key(42)
    sampled = jax.random.randint(skey, (B, H, SAMPLE_SIZE), 0, S)
    sampled = sampled.reshape(BH, SAMPLE_SIZE).astype(jnp.int32)

    q_s, k_s, v_s, ksub, vsub, sidx, rank = _sort_gather(
        codes_q, codes_k, qf, kf, vf, sampled)

    attn_sorted = _block_attention(
        q_s, k_s, v_s, ksub, vsub,
        sampled.reshape(BH, 1, SAMPLE_SIZE),
        sidx.reshape(BH * (S // BLOCK_SIZE), BLOCK_SIZE, 1),
        scale)

    out = _unsort(attn_sorted, rank)
    return out.reshape(B, H, S, D)


# speedup vs baseline: 7.6464x; 1.1497x over previous
import functools
import math

import jax
import jax.numpy as jnp
from jax import lax
from jax.experimental import pallas as pl
from jax.experimental.pallas import tpu as pltpu
from jax.experimental.pallas import tpu_sc as plsc

INPUT_DIM = 64
NUM_PROJS = 8
BLOCK_SIZE = 256
SAMPLE_SIZE = 256

NC = 2
NS = 16
LANES = 16
CHUNK = 128
NBUF = 4
PADW = 128


def _hash_body(q_ref, k_ref, proj_ref, cq_ref, ck_ref):
    proj = proj_ref[...]
    powers = (2 ** lax.iota(jnp.int32, NUM_PROJS))[None, :]

    def codes(x):
        s = lax.dot_general(x, proj, (((1,), (0,)), ((), ())),
                            preferred_element_type=jnp.float32)
        bits = (s > 0).astype(jnp.int32)
        return jnp.sum(bits * powers, axis=-1)

    cq_ref[0] = codes(q_ref[0]).reshape(cq_ref.shape[1:])
    ck_ref[0] = codes(k_ref[0]).reshape(ck_ref.shape[1:])


def _hash_codes(qf, kf, proj):
    BH, S, D = qf.shape
    return pl.pallas_call(
        _hash_body,
        grid=(BH,),
        in_specs=[
            pl.BlockSpec((1, S, D), lambda c: (c, 0, 0)),
            pl.BlockSpec((1, S, D), lambda c: (c, 0, 0)),
            pl.BlockSpec((D, NUM_PROJS), lambda c: (0, 0)),
        ],
        out_specs=[
            pl.BlockSpec((1, S // 128, 128), lambda c: (c, 0, 0)),
            pl.BlockSpec((1, S // 128, 128), lambda c: (c, 0, 0)),
        ],
        out_shape=[
            jax.ShapeDtypeStruct((BH, S // 128, 128), jnp.int32),
            jax.ShapeDtypeStruct((BH, S // 128, 128), jnp.int32),
        ],
    )(qf, kf, proj)


def _count_sort(cvm, hist, cur, rvm, svm, n):
    per_lane = n // LANES
    lane = lax.iota(jnp.int32, LANES)
    ones = jnp.ones((LANES,), jnp.int32)
    zeros = jnp.zeros((LANES,), jnp.int32)

    def zero_body(t, _):
        plsc.store_scatter(hist, [t * LANES + lane], zeros)
        return 0

    lax.fori_loop(0, 256, zero_body, 0)

    def p1_body(t, _):
        idx = lane * per_lane + t
        c16 = plsc.load_gather(cvm, [idx])
        plsc.addupdate_scatter(hist, [c16 * LANES + lane], ones)
        return 0

    lax.fori_loop(0, per_lane, p1_body, 0)

    def p2_body(c, carry):
        v = plsc.load_gather(hist, [c * LANES + lane])
        cs = plsc.cumsum(v)
        plsc.store_scatter(cur, [c * LANES + lane], carry + cs - v)
        return carry + jnp.sum(v)

    lax.fori_loop(0, 256, p2_body, jnp.int32(0))

    def p3_body(t, _):
        idx = lane * per_lane + t
        c16 = plsc.load_gather(cvm, [idx])
        slot = c16 * LANES + lane
        r16 = plsc.load_gather(cur, [slot])
        if rvm is not None:
            plsc.store_scatter(rvm, [idx], r16)
        plsc.store_scatter(svm, [r16], idx)
        plsc.addupdate_scatter(cur, [slot], ones)
        return 0

    lax.fori_loop(0, per_lane, p3_body, 0)


def _gather_rows(table, idxvm, outdst, chunks, bufs, sems_r, sems_w,
                 dst_cols=None, src_cols=None):
    nb = min(NBUF, chunks)
    steps = chunks // nb

    def step(s, _):
        handles = []
        for b in range(nb):
            c = s * nb + b
            idxr = idxvm.at[pl.ds(c * CHUNK, CHUNK)]
            handles.append(pltpu.async_copy(table.at[idxr], bufs.at[b],
                                            sems_r[b]))
        writes = []
        for b in range(nb):
            handles[b].wait()
            c = s * nb + b
            src = bufs.at[b]
            dst = outdst.at[pl.ds(c * CHUNK, CHUNK)]
            if src_cols is not None:
                src = src.at[:, pl.ds(0, src_cols)]
            if dst_cols is not None:
                dst = dst.at[:, pl.ds(0, dst_cols)]
            writes.append(pltpu.async_copy(src, dst, sems_w[b]))
        for wh in writes:
            wh.wait()
        return 0

    lax.fori_loop(0, steps, step, 0)


def _sort_gather(codes_q, codes_k, qf, kf, vf, sampled):
    BH, S, D = qf.shape
    mesh = plsc.VectorSubcoreMesh(core_axis_name="c", subcore_axis_name="s",
                                  num_cores=NC, num_subcores=NS)

    @functools.partial(
        pl.kernel,
        out_type=[
            jax.ShapeDtypeStruct((BH, S, PADW), jnp.float32),
            jax.ShapeDtypeStruct((BH, S, PADW), jnp.float32),
            jax.ShapeDtypeStruct((BH, S, PADW), jnp.float32),
            jax.ShapeDtypeStruct((BH, SAMPLE_SIZE, PADW), jnp.float32),
            jax.ShapeDtypeStruct((BH, SAMPLE_SIZE, PADW), jnp.float32),
            jax.ShapeDtypeStruct((BH, S), jnp.int32),
            jax.ShapeDtypeStruct((BH, S), jnp.int32),
        ],
        mesh=mesh,
        scratch_types=[
            pltpu.VMEM((S,), jnp.int32),
            pltpu.VMEM((256 * LANES,), jnp.int32),
            pltpu.VMEM((256 * LANES,), jnp.int32),
            pltpu.VMEM((S,), jnp.int32),
            pltpu.VMEM((S,), jnp.int32),
            pltpu.VMEM((S,), jnp.int32),
            pltpu.VMEM((SAMPLE_SIZE,), jnp.int32),
            pltpu.VMEM((NBUF, CHUNK, D), jnp.float32),
            pltpu.SemaphoreType.DMA,
            pltpu.SemaphoreType.DMA,
            pltpu.SemaphoreType.DMA,
            pltpu.SemaphoreType.DMA,
            pltpu.SemaphoreType.DMA,
            pltpu.SemaphoreType.DMA,
            pltpu.SemaphoreType.DMA,
            pltpu.SemaphoreType.DMA,
        ],
        compiler_params=pltpu.CompilerParams(needs_layout_passes=False, use_tc_tiling_on_sc=False),
    )
    def run(cq_hbm, ck_hbm, q_hbm, k_hbm, v_hbm, smp_hbm,
            qs_o, ks_o, vs_o, ksub_o, vsub_o, sidx_o, rank_o,
            cvm, hist, cur, rvm, svq, svk, smp, bufs,
            sr0, sr1, sr2, sr3, sw0, sw1, sw2, sw3):
        sems_r = [sr0, sr1, sr2, sr3]
        sems_w = [sw0, sw1, sw2, sw3]
        w = lax.axis_index("s") * NC + lax.axis_index("c")

        pltpu.sync_copy(cq_hbm.at[w], cvm)
        _count_sort(cvm, hist, cur, rvm, svq, S)
        pltpu.sync_copy(rvm, rank_o.at[w])
        pltpu.sync_copy(svq, sidx_o.at[w])

        pltpu.sync_copy(ck_hbm.at[w], cvm)
        _count_sort(cvm, hist, cur, None, svk, S)

        pltpu.sync_copy(smp_hbm.at[w], smp)

        nchunks = S // CHUNK
        _gather_rows(q_hbm.at[w], svq, qs_o.at[w], nchunks, bufs, sems_r,
                     sems_w, dst_cols=D)
        _gather_rows(k_hbm.at[w], svk, ks_o.at[w], nchunks, bufs, sems_r,
                     sems_w, dst_cols=D)
        _gather_rows(v_hbm.at[w], svk, vs_o.at[w], nchunks, bufs, sems_r,
                     sems_w, dst_cols=D)
        _gather_rows(k_hbm.at[w], smp, ksub_o.at[w], SAMPLE_SIZE // CHUNK,
                     bufs, sems_r, sems_w, dst_cols=D)
        _gather_rows(v_hbm.at[w], smp, vsub_o.at[w], SAMPLE_SIZE // CHUNK,
                     bufs, sems_r, sems_w, dst_cols=D)

    return run(codes_q, codes_k, qf, kf, vf, sampled)


def _unsort(attn_sorted, rank):
    BH, S, W = attn_sorted.shape
    D = INPUT_DIM
    mesh = plsc.VectorSubcoreMesh(core_axis_name="c", subcore_axis_name="s",
                                  num_cores=NC, num_subcores=NS)

    @functools.partial(
        pl.kernel,
        out_type=jax.ShapeDtypeStruct((BH, S, D), jnp.float32),
        mesh=mesh,
        scratch_types=[
            pltpu.VMEM((S,), jnp.int32),
            pltpu.VMEM((NBUF, CHUNK, W), jnp.float32),
            pltpu.SemaphoreType.DMA,
            pltpu.SemaphoreType.DMA,
            pltpu.SemaphoreType.DMA,
            pltpu.SemaphoreType.DMA,
            pltpu.SemaphoreType.DMA,
            pltpu.SemaphoreType.DMA,
            pltpu.SemaphoreType.DMA,
            pltpu.SemaphoreType.DMA,
        ],
        compiler_params=pltpu.CompilerParams(needs_layout_passes=False, use_tc_tiling_on_sc=False),
    )
    def run(a_hbm, r_hbm, out_hbm, rvm, bufs,
            sr0, sr1, sr2, sr3, sw0, sw1, sw2, sw3):
        w = lax.axis_index("s") * NC + lax.axis_index("c")
        pltpu.sync_copy(r_hbm.at[w], rvm)
        _gather_rows(a_hbm.at[w], rvm, out_hbm.at[w], S // CHUNK, bufs,
                     [sr0, sr1, sr2, sr3], [sw0, sw1, sw2, sw3],
                     src_cols=D)

    return run(attn_sorted, rank)


def _attn_body(scale, q_ref, k_ref, v_ref, ks_ref, vs_ref, sset_ref,
               sidx_ref, out_ref):
    D = INPUT_DIM
    q = q_ref[0][:, :D]
    k = k_ref[0][:, :D]
    v = v_ref[0][:, :D]
    ones_col = jnp.ones((BLOCK_SIZE, 1), jnp.float32)

    s1 = lax.dot_general(q, k, (((1,), (1,)), ((), ())),
                         preferred_element_type=jnp.float32) * scale
    m1 = jnp.max(s1, axis=-1, keepdims=True)
    e1 = jnp.exp(s1 - m1)
    l1 = lax.dot_general(e1, ones_col, (((1,), (0,)), ((), ())),
                         preferred_element_type=jnp.float32)
    o1 = lax.dot_general(e1, v, (((1,), (0,)), ((), ())),
                         preferred_element_type=jnp.float32) / l1
    lse1 = m1 + jnp.log(l1)

    ks = ks_ref[0][:, :D]
    vs = vs_ref[0][:, :D]
    qpos = sidx_ref[0]
    spos = sset_ref[0]
    mask = (qpos // BLOCK_SIZE) == (spos // BLOCK_SIZE)
    bias = jnp.where(mask, jnp.finfo(jnp.float32).min, jnp.float32(0.0))
    s2 = lax.dot_general(q, ks, (((1,), (1,)), ((), ())),
                         preferred_element_type=jnp.float32) * scale + bias
    m2 = jnp.max(s2, axis=-1, keepdims=True)
    e2 = jnp.exp(s2 - m2)
    l2 = lax.dot_general(e2, ones_col, (((1,), (0,)), ((), ())),
                         preferred_element_type=jnp.float32)
    o2 = lax.dot_general(e2, vs, (((1,), (0,)), ((), ())),
                         preferred_element_type=jnp.float32) / l2
    lse2 = m2 + jnp.log(l2) + math.log(16.0)

    c = 1.0 / (1.0 + jnp.exp(lse2 - lse1))
    out_ref[0, :, :D] = c * o1 + (1.0 - c) * o2


def _block_attention(q_s, k_s, v_s, ksub, vsub, sset3, sidx3, scale):
    BH, S, W = q_s.shape
    nb = S // BLOCK_SIZE
    return pl.pallas_call(
        functools.partial(_attn_body, scale),
        grid=(BH * nb,),
        in_specs=[
            pl.BlockSpec((1, BLOCK_SIZE, W), lambda c: (c // nb, c % nb, 0)),
            pl.BlockSpec((1, BLOCK_SIZE, W), lambda c: (c // nb, c % nb, 0)),
            pl.BlockSpec((1, BLOCK_SIZE, W), lambda c: (c // nb, c % nb, 0)),
            pl.BlockSpec((1, SAMPLE_SIZE, W), lambda c: (c // nb, 0, 0)),
            pl.BlockSpec((1, SAMPLE_SIZE, W), lambda c: (c // nb, 0, 0)),
            pl.BlockSpec((1, 1, SAMPLE_SIZE), lambda c: (c // nb, 0, 0)),
            pl.BlockSpec((1, BLOCK_SIZE, 1), lambda c: (c, 0, 0)),
        ],
        out_specs=pl.BlockSpec((1, BLOCK_SIZE, W),
                               lambda c: (c // nb, c % nb, 0)),
        out_shape=jax.ShapeDtypeStruct((BH, S, W), jnp.float32),
    )(q_s, k_s, v_s, ksub, vsub, sset3, sidx3)


def kernel(query, key, value, proj_dir):
    B, H, S, D = query.shape
    BH = B * H
    scale = D ** (-0.5)

    qf = query.reshape(BH, S, D)
    kf = key.reshape(BH, S, D)
    vf = value.reshape(BH, S, D)
    proj = proj_dir[0, 0]

    codes_q, codes_k = _hash_codes(qf, kf, proj)
    codes_q = codes_q.reshape(BH, S)
    codes_k = codes_k.reshape(BH, S)

    skey = jax.random.key(42)
    sampled = jax.random.randint(skey, (B, H, SAMPLE_SIZE), 0, S)
    sampled = sampled.reshape(BH, SAMPLE_SIZE).astype(jnp.int32)

    q_s, k_s, v_s, ksub, vsub, sidx, rank = _sort_gather(
        codes_q, codes_k, qf, kf, vf, sampled)

    attn_sorted = _block_attention(
        q_s, k_s, v_s, ksub, vsub,
        sampled.reshape(BH, 1, SAMPLE_SIZE),
        sidx.reshape(BH * (S // BLOCK_SIZE), BLOCK_SIZE, 1),
        scale)

    out = _unsort(attn_sorted, rank)
    return out.reshape(B, H, S, D)
